# baseline (speedup 1.0000x reference)
import jax
import jax.numpy as jnp
from jax import lax
from jax.experimental import pallas as pl
from jax.experimental.pallas import tpu as pltpu
from jax.experimental.pallas import tpu_sc as plsc

_TEMPERATURE = 2.0 / 3.0
_MAGIC = 0.8
_ROWS = 32
_COLS = 32768
_K = _COLS - 16384
_NVEC = _COLS // 16
_NBIN = 1024
_NGRP = _NBIN // 16


def _sig_body(z_ref, o_ref):
    x = z_ref[...] * jnp.float32(jnp.float32(1.5) * jnp.float32(_MAGIC))
    o_ref[...] = 1.0 / (1.0 + jnp.exp(-x))


def _sigmoid_tc(z):
    return pl.pallas_call(
        _sig_body,
        grid=(4,),
        in_specs=[pl.BlockSpec((8, _COLS), lambda i: (i, 0))],
        out_specs=pl.BlockSpec((8, _COLS), lambda i: (i, 0)),
        out_shape=jax.ShapeDtypeStruct((_ROWS, _COLS), jnp.float32),
    )(z)


def _swizzle(bin_, lane):
    return (bin_ << 4) | (lane ^ (bin_ & 15))


def _scan_level(hist, k_rem, rezero=True):
    lane = lax.iota(jnp.int32, 16)
    zeros16 = jnp.zeros((16,), jnp.int32)
    init = (zeros16, zeros16, zeros16)

    @plsc.parallel_loop(0, _NGRP, unroll=4, carry=init)
    def grp(g, carry):
        c0, nlt, cb = carry
        base = g * 256 + (lane << 4)
        tot = zeros16
        for l in range(16):
            idx = base + (lane ^ l)
            tot = tot + plsc.load_gather(hist, [idx])
            if rezero:
                plsc.store_scatter(hist, [idx], zeros16)
        cum = c0 + plsc.cumsum(tot)
        lt = cum < k_rem
        nlt = nlt + lt.astype(jnp.int32)
        cb = jnp.maximum(cb, jnp.where(lt, cum, 0))
        c0 = jnp.broadcast_to(cum[15], (16,))
        return c0, nlt, cb

    _, nlt, cb_v = grp
    b = jnp.sum(nlt)
    cb = jnp.max(cb_v)
    return b, cb, k_rem - cb


def _sc_body(s_hbm, out_hbm, s_v, hist, sem):
    nc = 2
    wid = lax.axis_index("s") * nc + lax.axis_index("c")
    copy_in = pltpu.async_copy(s_hbm.at[wid], s_v, sem)

    lane = lax.iota(jnp.int32, 16)
    ones = jnp.ones((16,), jnp.int32)
    zeros16 = jnp.zeros((16,), jnp.int32)

    @plsc.parallel_loop(0, _NBIN, unroll=8)
    def z_it(i):
        hist[pl.ds(i * 16, 16)] = zeros16

    copy_in.wait()

    scale = jnp.float32(jnp.float32(1.5) * jnp.float32(_MAGIC))

    @plsc.parallel_loop(0, _NVEC, unroll=8)
    def p0(i):
        z = s_v[pl.ds(i * 16, 16)]
        s = 1.0 / (1.0 + jnp.exp(-(z * scale)))
        s_v[pl.ds(i * 16, 16)] = s
        key = lax.bitcast_convert_type(s, jnp.int32)
        plsc.addupdate_scatter(hist, [_swizzle(key >> 20, lane)], ones)

    b1, cb1, k2 = _scan_level(hist, jnp.int32(_K))

    @plsc.parallel_loop(0, _NVEC, unroll=8)
    def p1(i):
        key = lax.bitcast_convert_type(s_v[pl.ds(i * 16, 16)], jnp.int32)
        m = (key >> 20) == b1
        plsc.addupdate_scatter(hist, [_swizzle((key >> 10) & 1023, lane)],
                               ones, mask=m)

    b2, cb2, k3 = _scan_level(hist, k2)

    pref = (b1 << 10) | b2

    @plsc.parallel_loop(0, _NVEC, unroll=8)
    def p2(i):
        key = lax.bitcast_convert_type(s_v[pl.ds(i * 16, 16)], jnp.int32)
        m = (key >> 10) == pref
        plsc.addupdate_scatter(hist, [_swizzle(key & 1023, lane)], ones,
                               mask=m)

    b3, cb3, m_eq = _scan_level(hist, k3, rezero=False)

    tbits = (pref << 10) | b3

    def p3_block(i, eq_seen):
        vs, keys, cums = [], [], []
        for j in range(16):
            v = s_v[pl.ds((i * 16 + j) * 16, 16)]
            key = lax.bitcast_convert_type(v, jnp.int32)
            vs.append(v)
            keys.append(key)
            cums.append(plsc.cumsum((key == tbits).astype(jnp.int32)))
        run = eq_seen
        for j in range(16):
            r = run + cums[j]
            zmask = (keys[j] < tbits) | ((keys[j] == tbits) & (r <= m_eq))
            s_v[pl.ds((i * 16 + j) * 16, 16)] = jnp.where(zmask, 0.0, vs[j])
            run = run + cums[j][15]
        return run

    half = _NVEC // 16 // 2
    eq_mid = plsc.parallel_loop(0, half, carry=jnp.int32(0))(p3_block)
    copy_lo = pltpu.async_copy(s_v.at[pl.ds(0, _COLS // 2)],
                               out_hbm.at[wid, pl.ds(0, _COLS // 2)], sem)
    plsc.parallel_loop(half, 2 * half, carry=eq_mid)(p3_block)
    copy_lo.wait()
    pltpu.sync_copy(s_v.at[pl.ds(_COLS // 2, _COLS // 2)],
                    out_hbm.at[wid, pl.ds(_COLS // 2, _COLS // 2)])


def _select_sc(s):
    mesh = plsc.VectorSubcoreMesh(core_axis_name="c", subcore_axis_name="s")
    return pl.kernel(
        _sc_body,
        mesh=mesh,
        compiler_params=pltpu.CompilerParams(needs_layout_passes=False),
        out_type=jax.ShapeDtypeStruct((_ROWS, _COLS), jnp.float32),
        scratch_types=[
            pltpu.VMEM((_COLS,), jnp.float32),
            pltpu.VMEM((_NBIN * 16,), jnp.int32),
            pltpu.SemaphoreType.DMA,
        ],
    )(s)


@jax.jit
def kernel(z_loga):
    return _select_sc(z_loga)


# speedup vs baseline: 1.1516x; 1.1516x over previous
import jax
import jax.numpy as jnp
from jax import lax
from jax.experimental import pallas as pl
from jax.experimental.pallas import tpu as pltpu
from jax.experimental.pallas import tpu_sc as plsc

_TEMPERATURE = 2.0 / 3.0
_MAGIC = 0.8
_ROWS = 32
_COLS = 32768
_K = _COLS - 16384
_NVEC = _COLS // 16
_NBIN = 1024
_NGRP = _NBIN // 16


def _sig_body(z_ref, o_ref):
    x = z_ref[...] * jnp.float32(jnp.float32(1.5) * jnp.float32(_MAGIC))
    o_ref[...] = 1.0 / (1.0 + jnp.exp(-x))


def _sigmoid_tc(z):
    return pl.pallas_call(
        _sig_body,
        grid=(4,),
        in_specs=[pl.BlockSpec((8, _COLS), lambda i: (i, 0))],
        out_specs=pl.BlockSpec((8, _COLS), lambda i: (i, 0)),
        out_shape=jax.ShapeDtypeStruct((_ROWS, _COLS), jnp.float32),
    )(z)


def _swizzle(bin_, lane):
    return (bin_ << 4) | (lane ^ (bin_ & 15))


def _scan_level(hist, k_rem, rezero=True):
    lane = lax.iota(jnp.int32, 16)
    zeros16 = jnp.zeros((16,), jnp.int32)
    init = (zeros16, zeros16, zeros16)

    @plsc.parallel_loop(0, _NGRP, unroll=4, carry=init)
    def grp(g, carry):
        c0, nlt, cb = carry
        base = g * 256 + (lane << 4)
        tot = zeros16
        for l in range(16):
            idx = base + (lane ^ l)
            tot = tot + plsc.load_gather(hist, [idx])
            if rezero:
                plsc.store_scatter(hist, [idx], zeros16)
        cum = c0 + plsc.cumsum(tot)
        lt = cum < k_rem
        nlt = nlt + lt.astype(jnp.int32)
        cb = jnp.maximum(cb, jnp.where(lt, cum, 0))
        c0 = jnp.broadcast_to(cum[15], (16,))
        return c0, nlt, cb

    _, nlt, cb_v = grp
    b = jnp.sum(nlt)
    cb = jnp.max(cb_v)
    return b, cb, k_rem - cb


def _sc_body(s_hbm, out_hbm, s_v, hist, sem):
    nc = 2
    wid = lax.axis_index("s") * nc + lax.axis_index("c")
    copy_in = pltpu.async_copy(s_hbm.at[wid], s_v, sem)

    lane = lax.iota(jnp.int32, 16)
    ones = jnp.ones((16,), jnp.int32)
    zeros16 = jnp.zeros((16,), jnp.int32)

    @plsc.parallel_loop(0, _NBIN, unroll=8)
    def z_it(i):
        hist[pl.ds(i * 16, 16)] = zeros16

    copy_in.wait()

    scale = jnp.float32(jnp.float32(1.5) * jnp.float32(_MAGIC))

    @plsc.parallel_loop(0, _NVEC, unroll=8)
    def p0(i):
        z = s_v[pl.ds(i * 16, 16)]
        s = 1.0 / (1.0 + jnp.exp(-(z * scale)))
        s_v[pl.ds(i * 16, 16)] = s
        key = lax.bitcast_convert_type(s, jnp.int32)
        plsc.addupdate_scatter(hist, [_swizzle(key >> 20, lane)], ones)

    b1, cb1, k2 = _scan_level(hist, jnp.int32(_K))

    @plsc.parallel_loop(0, _NVEC, unroll=8)
    def p1(i):
        key = lax.bitcast_convert_type(s_v[pl.ds(i * 16, 16)], jnp.int32)
        m = (key >> 20) == b1
        plsc.addupdate_scatter(hist, [_swizzle((key >> 10) & 1023, lane)],
                               ones, mask=m)

    b2, cb2, k3 = _scan_level(hist, k2)

    pref = (b1 << 10) | b2

    @plsc.parallel_loop(0, _NVEC, unroll=8)
    def p2(i):
        key = lax.bitcast_convert_type(s_v[pl.ds(i * 16, 16)], jnp.int32)
        m = (key >> 10) == pref
        plsc.addupdate_scatter(hist, [_swizzle(key & 1023, lane)], ones,
                               mask=m)

    b3, cb3, m_eq = _scan_level(hist, k3, rezero=False)

    tbits = (pref << 10) | b3

    def p3_block(i, eq_seen):
        vs, keys, cums = [], [], []
        for j in range(8):
            v = s_v[pl.ds((i * 8 + j) * 16, 16)]
            key = lax.bitcast_convert_type(v, jnp.int32)
            vs.append(v)
            keys.append(key)
            cums.append(plsc.cumsum((key == tbits).astype(jnp.int32)))
        run = eq_seen
        for j in range(8):
            r = run + cums[j]
            zmask = (keys[j] < tbits) | ((keys[j] == tbits) & (r <= m_eq))
            s_v[pl.ds((i * 8 + j) * 16, 16)] = jnp.where(zmask, 0.0, vs[j])
            run = run + cums[j][15]
        return run

    half = _NVEC // 8 // 2
    eq_mid = plsc.parallel_loop(0, half, carry=jnp.int32(0))(p3_block)
    copy_lo = pltpu.async_copy(s_v.at[pl.ds(0, _COLS // 2)],
                               out_hbm.at[wid, pl.ds(0, _COLS // 2)], sem)
    plsc.parallel_loop(half, 2 * half, carry=eq_mid)(p3_block)
    copy_lo.wait()
    pltpu.sync_copy(s_v.at[pl.ds(_COLS // 2, _COLS // 2)],
                    out_hbm.at[wid, pl.ds(_COLS // 2, _COLS // 2)])


def _select_sc(s):
    mesh = plsc.VectorSubcoreMesh(core_axis_name="c", subcore_axis_name="s")
    return pl.kernel(
        _sc_body,
        mesh=mesh,
        compiler_params=pltpu.CompilerParams(needs_layout_passes=False),
        out_type=jax.ShapeDtypeStruct((_ROWS, _COLS), jnp.float32),
        scratch_types=[
            pltpu.VMEM((_COLS,), jnp.float32),
            pltpu.VMEM((_NBIN * 16,), jnp.int32),
            pltpu.SemaphoreType.DMA,
        ],
    )(s)


@jax.jit
def kernel(z_loga):
    return _select_sc(z_loga)
